# baseline (speedup 1.0000x reference)
import functools

import numpy as np
import jax
import jax.numpy as jnp
from jax import lax
from jax.experimental import pallas as pl
from jax.experimental.pallas import tpu as pltpu
from jax.experimental.pallas import tpu_sc as plsc

_NUM_BINS = 1024
_NUM_TILINGS = 16
_NUM_DIMS = 2
_NUM_EDGES = _NUM_BINS + 1
_LIMITS = np.array([[0.0, 1.0], [0.0, 1.0]], dtype=np.float64)


def _make_edges_lane_major():
    edges = np.zeros((_NUM_TILINGS, _NUM_EDGES), dtype=np.float64)
    for tiling in range(_NUM_TILINGS):
        dim_range = _LIMITS[0, 1] - _LIMITS[0, 0]
        bin_size = dim_range / (_NUM_BINS + (1.0 / _NUM_TILINGS - 1.0))
        tiling_range = dim_range + bin_size * (1.0 - 1.0 / _NUM_TILINGS)
        tiling_low = _LIMITS[0, 0] - bin_size * tiling / _NUM_TILINGS
        tiling_high = tiling_low + tiling_range
        edges[tiling, :] = np.linspace(tiling_low, tiling_high,
                                       num=_NUM_EDGES)
    edges32 = edges.astype(np.float32)
    return jnp.asarray(np.transpose(edges32, (1, 0)).reshape(-1))


_EDGES_SC = _make_edges_lane_major()
_INV_STEP = np.float32(
    1.0 / (1.0 / (_NUM_BINS + (1.0 / _NUM_TILINGS - 1.0))))


def _build_sc_call():
    mesh = plsc.VectorSubcoreMesh(core_axis_name="c", subcore_axis_name="s")

    @functools.partial(
        pl.kernel,
        mesh=mesh,
        compiler_params=pltpu.CompilerParams(needs_layout_passes=False),
        out_type=jax.ShapeDtypeStruct((16,), jnp.float32),
        scratch_types=[
            pltpu.VMEM((_NUM_DIMS, 16), jnp.float32),
            pltpu.VMEM((_NUM_EDGES * 16,), jnp.float32),
            pltpu.VMEM((16,), jnp.int32),
            pltpu.VMEM((16, _NUM_BINS), jnp.float32),
            pltpu.VMEM((16,), jnp.float32),
            pltpu.VMEM_SHARED((16,), jnp.int32),
            pltpu.SemaphoreType.DMA,
        ],
    )
    def tile_coding_sc(state_hbm, edges_hbm, w_hbm, out_hbm,
                       state_v, edges_v, idx_v, rows_v, res_v, echo_v, sem):
        cid = lax.axis_index("c")
        sid = lax.axis_index("s")

        @pl.when(jnp.logical_and(cid == 0, sid == 0))
        def _only():
            pltpu.sync_copy(state_hbm, state_v)
            pltpu.sync_copy(edges_hbm, edges_v)
            lane = lax.iota(jnp.int32, 16)
            xs = [state_v[d] for d in range(_NUM_DIMS)]
            e0 = plsc.load_gather(edges_v, [lane])
            los, his = [], []
            for d in range(_NUM_DIMS):
                k_est = ((xs[d] - e0) * _INV_STEP).astype(jnp.int32)
                w = jnp.clip(k_est - 4, 0, _NUM_EDGES - 15)
                los.append(w)
                his.append(w + 15)
            for _ in range(4):
                for d in range(_NUM_DIMS):
                    mid = lax.shift_right_arithmetic(los[d] + his[d], 1)
                    flat = mid * 16 + lane
                    ev = plsc.load_gather(edges_v, [flat])
                    le = ev <= xs[d]
                    los[d] = jnp.where(le, mid + 1, los[d])
                    his[d] = jnp.where(le, his[d], mid)
            bi = jnp.clip(los[0] - 1, 0, _NUM_BINS - 1)
            bj = jnp.clip(los[1] - 1, 0, _NUM_BINS - 1)
            idx_v[...] = lane * _NUM_BINS + bi
            pltpu.sync_copy(idx_v, echo_v)
            pltpu.async_copy(w_hbm.at[idx_v], rows_v, sem).wait()
            got = plsc.load_gather(rows_v, [lane, bj])
            res_v[...] = jnp.full((16,), jnp.sum(got), jnp.float32)
            pltpu.sync_copy(res_v, out_hbm)

    return tile_coding_sc


_SC_CALL_CACHE = []


def kernel(state, weights):
    if not _SC_CALL_CACHE:
        _SC_CALL_CACHE.append(_build_sc_call())
    state_b = jnp.broadcast_to(state[:, None], (_NUM_DIMS, 16))
    w_rows = weights.reshape(_NUM_TILINGS * _NUM_BINS, _NUM_BINS)
    out16 = _SC_CALL_CACHE[0](state_b, _EDGES_SC, w_rows)
    return out16[0]
